# baseline (speedup 1.0000x reference)
import jax
import jax.numpy as jnp
from jax import lax
from jax.experimental import pallas as pl
from jax.experimental.pallas import tpu as pltpu
import jax.experimental.pallas.tpu_sc as plsc

N = 10000
D = 128
E = 320000
B = 4096

NC = 2
NS = 16
NW = NC * NS
LANES = 16

K = 64
CPW = 160
CHUNKS = NW * CPW
EPAD = CHUNKS * K
NB = 4
NGRP = CPW // NB
ZR = 40
ZCH = N // ZR
ZITERS = -(-ZCH // NS)
FR = 80
FCH = N // FR
FITERS = -(-FCH // NS)
DEGW = 128
DK = 128
DEG_ITERS = -(-(E // DK) // NW)

TB = 4 * B
GPW = TB // NW
GK = 128
GCH = GPW // GK


def _mesh():
    return plsc.VectorSubcoreMesh(core_axis_name="c", subcore_axis_name="s",
                                  num_cores=NC, num_subcores=NS)


def _zero_2d(ref, rows, width):
    zero = jnp.zeros((LANES,), jnp.float32)

    def body(i, carry):
        for cb in range(width // LANES):
            ref[i, pl.ds(cb * LANES, LANES)] = zero
        return carry

    lax.fori_loop(0, rows, body, 0)


def _fill_ones(ref, rows, width):
    one = jnp.ones((LANES,), jnp.float32)

    def body(i, carry):
        for cb in range(width // LANES):
            ref[i, pl.ds(cb * LANES, LANES)] = one
        return carry

    lax.fori_loop(0, rows, body, 0)


def _sc_segsum_body(y_hbm, src_hbm, dst_hbm, za_out, zb_out,
                    srci_v, dsti_v, rows_v, acc_s,
                    sem_is, sem_id, sem_g, sem_s):
    c = lax.axis_index("c")
    s = lax.axis_index("s")
    wid = s * NC + c

    def issue_idx(j, iset, b):
        base = pl.multiple_of((wid * CPW + j) * K, 8)
        pltpu.async_copy(src_hbm.at[pl.ds(base, K)], srci_v.at[iset, b],
                         sem_is.at[iset, b])
        pltpu.async_copy(dst_hbm.at[pl.ds(base, K)], dsti_v.at[iset, b],
                         sem_id.at[iset, b])

    def wait_idx(j, iset, b):
        base = pl.multiple_of((wid * CPW + j) * K, 8)
        pltpu.make_async_copy(src_hbm.at[pl.ds(base, K)], srci_v.at[iset, b],
                              sem_is.at[iset, b]).wait()
        pltpu.make_async_copy(dst_hbm.at[pl.ds(base, K)], dsti_v.at[iset, b],
                              sem_id.at[iset, b]).wait()

    _zero_2d(rows_v.at[0], K, D)
    for j in range(ZITERS):
        zid = s + j * NS

        @pl.when(zid < ZCH)
        def _():
            r0 = pl.multiple_of(zid * ZR, 8)
            pltpu.sync_copy(rows_v.at[0, pl.ds(0, ZR)], acc_s.at[pl.ds(r0, ZR)])
    plsc.subcore_barrier()

    for b in range(NB):
        issue_idx(b, 0, b)

    def dual(gg, carry):
        for half in range(2):
            g = 2 * gg + half

            @pl.when(g > 0)
            def _():
                for b in range(NB):
                    pltpu.make_async_copy(
                        rows_v.at[b], acc_s.at[dsti_v.at[1 - half, b]],
                        sem_s.at[b]).wait()

            gds = []
            for b in range(NB):
                j = g * NB + b
                wait_idx(j, half, b)
                gds.append(pltpu.async_copy(
                    y_hbm.at[srci_v.at[half, b]], rows_v.at[b], sem_g.at[b]))

            @pl.when(g + 1 < NGRP)
            def _():
                for b in range(NB):
                    issue_idx((g + 1) * NB + b, 1 - half, b)

            for b in range(NB):
                gds[b].wait()
                pltpu.async_copy(rows_v.at[b], acc_s.at[dsti_v.at[half, b]],
                                 sem_s.at[b], add=True)
        return carry

    lax.fori_loop(0, NGRP // 2, dual, 0)
    for b in range(NB):
        pltpu.make_async_copy(
            rows_v.at[b], acc_s.at[dsti_v.at[1, b]], sem_s.at[b]).wait()
    plsc.subcore_barrier()

    for j in range(FITERS):
        fid = s + j * NS

        @pl.when(fid < FCH)
        def _():
            r0 = pl.multiple_of(fid * FR, 8)

            @pl.when(c == 0)
            def _():
                pltpu.sync_copy(acc_s.at[pl.ds(r0, FR)],
                                za_out.at[pl.ds(r0, FR)])

            @pl.when(c == 1)
            def _():
                pltpu.sync_copy(acc_s.at[pl.ds(r0, FR)],
                                zb_out.at[pl.ds(r0, FR)])


def _sc_segsum(y, srcp, dstp):
    zshape = jax.ShapeDtypeStruct((N, D), jnp.float32)
    k = pl.kernel(
        _sc_segsum_body,
        out_type=[zshape, zshape],
        mesh=_mesh(),
        scratch_types=[
            pltpu.VMEM((2, NB, K), jnp.int32),
            pltpu.VMEM((2, NB, K), jnp.int32),
            pltpu.VMEM((NB, K, D), jnp.float32),
            pltpu.VMEM_SHARED((N, D), jnp.float32),
            pltpu.SemaphoreType.DMA((2, NB)),
            pltpu.SemaphoreType.DMA((2, NB)),
            pltpu.SemaphoreType.DMA((NB,)),
            pltpu.SemaphoreType.DMA((NB,)),
        ],
    )
    return k(y, srcp, dstp)


def _sc_deg_body(dst_hbm, dega_out, degb_out,
                 dst_v, ones_v, zdeg_v, deg_s):
    c = lax.axis_index("c")
    s = lax.axis_index("s")
    wid = s * NC + c

    _zero_2d(zdeg_v, FR, DEGW)
    _fill_ones(ones_v, DK, DEGW)
    for j in range(FITERS):
        fid = s + j * NS

        @pl.when(fid < FCH)
        def _():
            r0 = pl.multiple_of(fid * FR, 8)
            pltpu.sync_copy(zdeg_v, deg_s.at[pl.ds(r0, FR)])
    plsc.subcore_barrier()

    def body(i, carry):
        cid = wid + i * NW

        @pl.when(cid < E // DK)
        def _():
            base = pl.multiple_of(cid * DK, 8)
            pltpu.sync_copy(dst_hbm.at[pl.ds(base, DK)], dst_v)
            pltpu.sync_copy(ones_v, deg_s.at[dst_v], add=True)

        return carry

    lax.fori_loop(0, DEG_ITERS, body, 0)
    plsc.subcore_barrier()

    for j in range(FITERS):
        fid = s + j * NS

        @pl.when(fid < FCH)
        def _():
            r0 = pl.multiple_of(fid * FR, 8)

            @pl.when(c == 0)
            def _():
                pltpu.sync_copy(deg_s.at[pl.ds(r0, FR)],
                                dega_out.at[pl.ds(r0, FR)])

            @pl.when(c == 1)
            def _():
                pltpu.sync_copy(deg_s.at[pl.ds(r0, FR)],
                                degb_out.at[pl.ds(r0, FR)])


def _sc_deg(dst):
    dshape = jax.ShapeDtypeStruct((N, DEGW), jnp.float32)
    k = pl.kernel(
        _sc_deg_body,
        out_type=[dshape, dshape],
        mesh=_mesh(),
        scratch_types=[
            pltpu.VMEM((DK,), jnp.int32),
            pltpu.VMEM((DK, DEGW), jnp.float32),
            pltpu.VMEM((FR, DEGW), jnp.float32),
            pltpu.VMEM_SHARED((N, DEGW), jnp.float32),
        ],
    )
    return k(dst)


def _sc_gather_body(h_hbm, idx_hbm, out_hbm, idx_v, rows_v, sem):
    c = lax.axis_index("c")
    s = lax.axis_index("s")
    wid = s * NC + c
    for j in range(GCH):
        base = pl.multiple_of(wid * GPW + j * GK, 8)
        pltpu.sync_copy(idx_hbm.at[pl.ds(base, GK)], idx_v)
        pltpu.async_copy(h_hbm.at[idx_v], rows_v, sem).wait()
        pltpu.sync_copy(rows_v, out_hbm.at[pl.ds(base, GK)])


def _sc_gather(h, idx):
    k = pl.kernel(
        _sc_gather_body,
        out_type=jax.ShapeDtypeStruct((TB, D), jnp.float32),
        mesh=_mesh(),
        scratch_types=[
            pltpu.VMEM((GK,), jnp.int32),
            pltpu.VMEM((GK, D), jnp.float32),
            pltpu.SemaphoreType.DMA,
        ],
    )
    return k(h, idx)


TC_R = 1000


def _tc1_body(x_ref, ws_ref, wn_ref, b_ref, a1_ref, y1_ref):
    xv = x_ref[...]
    a1_ref[...] = (jnp.dot(xv, ws_ref[...], preferred_element_type=jnp.float32)
                   + b_ref[...])
    y1_ref[...] = jnp.dot(xv, wn_ref[...], preferred_element_type=jnp.float32)


def _tc1(x, ws, wn, b):
    row_spec = pl.BlockSpec((TC_R, D), lambda i: (i, 0))
    w_spec = pl.BlockSpec((D, D), lambda i: (0, 0))
    b_spec = pl.BlockSpec((1, D), lambda i: (0, 0))
    return pl.pallas_call(
        _tc1_body,
        grid=(N // TC_R,),
        in_specs=[row_spec, w_spec, w_spec, b_spec],
        out_specs=[row_spec, row_spec],
        out_shape=[jax.ShapeDtypeStruct((N, D), jnp.float32)] * 2,
    )(x, ws, wn, b.reshape(1, D))


def _tc2_body(a1_ref, za_ref, zb_ref, da_ref, db_ref, ws_ref, wn_ref, b_ref,
              a2_ref, y2_ref):
    deg = da_ref[...] + db_ref[...]
    inv = 1.0 / jnp.maximum(deg[:, 0:1], 1.0)
    h1 = jnp.maximum(a1_ref[...] + (za_ref[...] + zb_ref[...]) * inv, 0.0)
    a2_ref[...] = (jnp.dot(h1, ws_ref[...], preferred_element_type=jnp.float32)
                   + b_ref[...])
    y2_ref[...] = jnp.dot(h1, wn_ref[...], preferred_element_type=jnp.float32)


def _tc2(a1, za, zb, da, db, ws, wn, b):
    row_spec = pl.BlockSpec((TC_R, D), lambda i: (i, 0))
    deg_spec = pl.BlockSpec((TC_R, DEGW), lambda i: (i, 0))
    w_spec = pl.BlockSpec((D, D), lambda i: (0, 0))
    b_spec = pl.BlockSpec((1, D), lambda i: (0, 0))
    return pl.pallas_call(
        _tc2_body,
        grid=(N // TC_R,),
        in_specs=[row_spec, row_spec, row_spec, deg_spec, deg_spec,
                  w_spec, w_spec, b_spec],
        out_specs=[row_spec, row_spec],
        out_shape=[jax.ShapeDtypeStruct((N, D), jnp.float32)] * 2,
    )(a1, za, zb, da, db, ws, wn, b.reshape(1, D))


def _tc3_body(a2_ref, za_ref, zb_ref, da_ref, db_ref, h2_ref):
    deg = da_ref[...] + db_ref[...]
    inv = 1.0 / jnp.maximum(deg[:, 0:1], 1.0)
    h2_ref[...] = a2_ref[...] + (za_ref[...] + zb_ref[...]) * inv


def _tc3(a2, za, zb, da, db):
    row_spec = pl.BlockSpec((TC_R, D), lambda i: (i, 0))
    deg_spec = pl.BlockSpec((TC_R, DEGW), lambda i: (i, 0))
    return pl.pallas_call(
        _tc3_body,
        grid=(N // TC_R,),
        in_specs=[row_spec, row_spec, row_spec, deg_spec, deg_spec],
        out_specs=row_spec,
        out_shape=jax.ShapeDtypeStruct((N, D), jnp.float32),
    )(a2, za, zb, da, db)


def kernel(x, edge_index, pos_src_idx, pos_dst_idx, neg_src_idx, neg_dst_idx,
           W_self1, W_neigh1, b1, W_self2, W_neigh2, b2):
    src = edge_index[0]
    dst = edge_index[1]
    npad = EPAD - E
    srcp = jnp.concatenate([src, jnp.full((npad,), N, src.dtype)])
    dstp = jnp.concatenate(
        [dst, (jnp.arange(npad, dtype=dst.dtype) * 131) % N])
    zrows = jnp.zeros((8, D), jnp.float32)

    dega, degb = _sc_deg(dst)
    a1, y1 = _tc1(x, W_self1, W_neigh1, b1)
    z1a, z1b = _sc_segsum(jnp.concatenate([y1, zrows]), srcp, dstp)
    a2, y2 = _tc2(a1, z1a, z1b, dega, degb, W_self2, W_neigh2, b2)
    z2a, z2b = _sc_segsum(jnp.concatenate([y2, zrows]), srcp, dstp)
    h2 = _tc3(a2, z2a, z2b, dega, degb)

    cat_idx = jnp.concatenate(
        [pos_src_idx, pos_dst_idx, neg_src_idx, neg_dst_idx])
    out = _sc_gather(h2, cat_idx)
    return (out[0:B], out[B:2 * B], out[2 * B:3 * B], out[3 * B:4 * B])


# speedup vs baseline: 2.2984x; 2.2984x over previous
import jax
import jax.numpy as jnp
from jax import lax
from jax.experimental import pallas as pl
from jax.experimental.pallas import tpu as pltpu
import jax.experimental.pallas.tpu_sc as plsc

N = 10000
D = 128
E = 320000
B = 4096

NC = 2
NS = 16
NW = NC * NS
LANES = 16

K = 64
CPW = 160
CHUNKS = NW * CPW
EPAD = CHUNKS * K
NB = 4
NGRP = CPW // NB
ZR = 40
ZCH = N // ZR
ZITERS = -(-ZCH // NS)
FR = 80
FCH = N // FR
FITERS = -(-FCH // NS)
DEGW = 128
DK = 128
DEG_ITERS = -(-(E // DK) // NW)

TB = 4 * B
GPW = TB // NW
GK = 128
GCH = GPW // GK


def _mesh():
    return plsc.VectorSubcoreMesh(core_axis_name="c", subcore_axis_name="s",
                                  num_cores=NC, num_subcores=NS)


def _zero_2d(ref, rows, width):
    zero = jnp.zeros((LANES,), jnp.float32)

    def body(i, carry):
        for cb in range(width // LANES):
            ref[i, pl.ds(cb * LANES, LANES)] = zero
        return carry

    lax.fori_loop(0, rows, body, 0)


def _fill_ones(ref, rows, width):
    one = jnp.ones((LANES,), jnp.float32)

    def body(i, carry):
        for cb in range(width // LANES):
            ref[i, pl.ds(cb * LANES, LANES)] = one
        return carry

    lax.fori_loop(0, rows, body, 0)


def _sc_segsum_body(y_hbm, src_hbm, dst_hbm, za_out, zb_out,
                    srci_v, dsti_v, rows_v, acc_s,
                    sem_is, sem_id, sem_g, sem_s):
    c = lax.axis_index("c")
    s = lax.axis_index("s")
    wid = s * NC + c

    def issue_idx(j, iset, b):
        base = pl.multiple_of((wid * CPW + j) * K, 8)
        pltpu.async_copy(src_hbm.at[pl.ds(base, K)], srci_v.at[iset, b],
                         sem_is.at[iset, b])
        pltpu.async_copy(dst_hbm.at[pl.ds(base, K)], dsti_v.at[iset, b],
                         sem_id.at[iset, b])

    def wait_idx(j, iset, b):
        base = pl.multiple_of((wid * CPW + j) * K, 8)
        pltpu.make_async_copy(src_hbm.at[pl.ds(base, K)], srci_v.at[iset, b],
                              sem_is.at[iset, b]).wait()
        pltpu.make_async_copy(dst_hbm.at[pl.ds(base, K)], dsti_v.at[iset, b],
                              sem_id.at[iset, b]).wait()

    _zero_2d(rows_v.at[0], K, D)
    for j in range(ZITERS):
        zid = s + j * NS

        @pl.when(zid < ZCH)
        def _():
            r0 = pl.multiple_of(zid * ZR, 8)
            pltpu.sync_copy(rows_v.at[0, pl.ds(0, ZR)], acc_s.at[pl.ds(r0, ZR)])
    plsc.subcore_barrier()

    for b in range(NB):
        issue_idx(b, 0, b)

    def dual(gg, carry):
        for half in range(2):
            g = 2 * gg + half

            @pl.when(g > 0)
            def _():
                for b in range(NB):
                    pltpu.make_async_copy(
                        rows_v.at[b], acc_s.at[dsti_v.at[1 - half, b]],
                        sem_s.at[b]).wait()

            gds = []
            for b in range(NB):
                j = g * NB + b
                wait_idx(j, half, b)
                gds.append(pltpu.async_copy(
                    y_hbm.at[srci_v.at[half, b]], rows_v.at[b], sem_g.at[b]))

            @pl.when(g + 1 < NGRP)
            def _():
                for b in range(NB):
                    issue_idx((g + 1) * NB + b, 1 - half, b)

            for b in range(NB):
                gds[b].wait()
                pltpu.async_copy(rows_v.at[b], acc_s.at[dsti_v.at[half, b]],
                                 sem_s.at[b], add=True)
        return carry

    lax.fori_loop(0, NGRP // 2, dual, 0)
    for b in range(NB):
        pltpu.make_async_copy(
            rows_v.at[b], acc_s.at[dsti_v.at[1, b]], sem_s.at[b]).wait()
    plsc.subcore_barrier()

    for j in range(FITERS):
        fid = s + j * NS

        @pl.when(fid < FCH)
        def _():
            r0 = pl.multiple_of(fid * FR, 8)

            @pl.when(c == 0)
            def _():
                pltpu.sync_copy(acc_s.at[pl.ds(r0, FR)],
                                za_out.at[pl.ds(r0, FR)])

            @pl.when(c == 1)
            def _():
                pltpu.sync_copy(acc_s.at[pl.ds(r0, FR)],
                                zb_out.at[pl.ds(r0, FR)])


def _sc_segsum(y, srcp, dstp):
    zshape = jax.ShapeDtypeStruct((N, D), jnp.float32)
    k = pl.kernel(
        _sc_segsum_body,
        out_type=[zshape, zshape],
        mesh=_mesh(),
        scratch_types=[
            pltpu.VMEM((2, NB, K), jnp.int32),
            pltpu.VMEM((2, NB, K), jnp.int32),
            pltpu.VMEM((NB, K, D), jnp.float32),
            pltpu.VMEM_SHARED((N, D), jnp.float32),
            pltpu.SemaphoreType.DMA((2, NB)),
            pltpu.SemaphoreType.DMA((2, NB)),
            pltpu.SemaphoreType.DMA((NB,)),
            pltpu.SemaphoreType.DMA((NB,)),
        ],
    )
    return k(y, srcp, dstp)


def _sc_deg_body(dst_hbm, dega_out, degb_out,
                 dst_v, ones_v, zdeg_v, deg_s):
    c = lax.axis_index("c")
    s = lax.axis_index("s")
    wid = s * NC + c

    _zero_2d(zdeg_v, FR, DEGW)
    _fill_ones(ones_v, DK, DEGW)
    for j in range(FITERS):
        fid = s + j * NS

        @pl.when(fid < FCH)
        def _():
            r0 = pl.multiple_of(fid * FR, 8)
            pltpu.sync_copy(zdeg_v, deg_s.at[pl.ds(r0, FR)])
    plsc.subcore_barrier()

    def body(i, carry):
        cid = wid + i * NW

        @pl.when(cid < E // DK)
        def _():
            base = pl.multiple_of(cid * DK, 8)
            pltpu.sync_copy(dst_hbm.at[pl.ds(base, DK)], dst_v)
            pltpu.sync_copy(ones_v, deg_s.at[dst_v], add=True)

        return carry

    lax.fori_loop(0, DEG_ITERS, body, 0)
    plsc.subcore_barrier()

    for j in range(FITERS):
        fid = s + j * NS

        @pl.when(fid < FCH)
        def _():
            r0 = pl.multiple_of(fid * FR, 8)

            @pl.when(c == 0)
            def _():
                pltpu.sync_copy(deg_s.at[pl.ds(r0, FR)],
                                dega_out.at[pl.ds(r0, FR)])

            @pl.when(c == 1)
            def _():
                pltpu.sync_copy(deg_s.at[pl.ds(r0, FR)],
                                degb_out.at[pl.ds(r0, FR)])


def _sc_deg(dst):
    dshape = jax.ShapeDtypeStruct((N, DEGW), jnp.float32)
    k = pl.kernel(
        _sc_deg_body,
        out_type=[dshape, dshape],
        mesh=_mesh(),
        scratch_types=[
            pltpu.VMEM((DK,), jnp.int32),
            pltpu.VMEM((DK, DEGW), jnp.float32),
            pltpu.VMEM((FR, DEGW), jnp.float32),
            pltpu.VMEM_SHARED((N, DEGW), jnp.float32),
        ],
    )
    return k(dst)


def _sc_gather_body(h_hbm, idx_hbm, out_hbm, idx_v, rows_v, sem):
    c = lax.axis_index("c")
    s = lax.axis_index("s")
    wid = s * NC + c
    for j in range(GCH):
        base = pl.multiple_of(wid * GPW + j * GK, 8)
        pltpu.sync_copy(idx_hbm.at[pl.ds(base, GK)], idx_v)
        pltpu.async_copy(h_hbm.at[idx_v], rows_v, sem).wait()
        pltpu.sync_copy(rows_v, out_hbm.at[pl.ds(base, GK)])


def _sc_gather(h, idx):
    k = pl.kernel(
        _sc_gather_body,
        out_type=jax.ShapeDtypeStruct((TB, D), jnp.float32),
        mesh=_mesh(),
        scratch_types=[
            pltpu.VMEM((GK,), jnp.int32),
            pltpu.VMEM((GK, D), jnp.float32),
            pltpu.SemaphoreType.DMA,
        ],
    )
    return k(h, idx)


TC_R = 1000


def _tc1_body(x_ref, ws_ref, wn_ref, b_ref, a1_ref, y1_ref):
    xv = x_ref[...]
    a1_ref[...] = (jnp.dot(xv, ws_ref[...], preferred_element_type=jnp.float32)
                   + b_ref[...])
    y1_ref[...] = jnp.dot(xv, wn_ref[...], preferred_element_type=jnp.float32)


def _tc1(x, ws, wn, b):
    row_spec = pl.BlockSpec((TC_R, D), lambda i: (i, 0))
    w_spec = pl.BlockSpec((D, D), lambda i: (0, 0))
    b_spec = pl.BlockSpec((1, D), lambda i: (0, 0))
    return pl.pallas_call(
        _tc1_body,
        grid=(N // TC_R,),
        in_specs=[row_spec, w_spec, w_spec, b_spec],
        out_specs=[row_spec, row_spec],
        out_shape=[jax.ShapeDtypeStruct((N, D), jnp.float32)] * 2,
    )(x, ws, wn, b.reshape(1, D))


def _tc2_body(a1_ref, za_ref, zb_ref, da_ref, db_ref, ws_ref, wn_ref, b_ref,
              a2_ref, y2_ref):
    deg = da_ref[...] + db_ref[...]
    inv = 1.0 / jnp.maximum(deg[:, 0:1], 1.0)
    h1 = jnp.maximum(a1_ref[...] + (za_ref[...] + zb_ref[...]) * inv, 0.0)
    a2_ref[...] = (jnp.dot(h1, ws_ref[...], preferred_element_type=jnp.float32)
                   + b_ref[...])
    y2_ref[...] = jnp.dot(h1, wn_ref[...], preferred_element_type=jnp.float32)


def _tc2(a1, za, zb, da, db, ws, wn, b):
    row_spec = pl.BlockSpec((TC_R, D), lambda i: (i, 0))
    deg_spec = pl.BlockSpec((TC_R, DEGW), lambda i: (i, 0))
    w_spec = pl.BlockSpec((D, D), lambda i: (0, 0))
    b_spec = pl.BlockSpec((1, D), lambda i: (0, 0))
    return pl.pallas_call(
        _tc2_body,
        grid=(N // TC_R,),
        in_specs=[row_spec, row_spec, row_spec, deg_spec, deg_spec,
                  w_spec, w_spec, b_spec],
        out_specs=[row_spec, row_spec],
        out_shape=[jax.ShapeDtypeStruct((N, D), jnp.float32)] * 2,
    )(a1, za, zb, da, db, ws, wn, b.reshape(1, D))


def _tc3_body(a2_ref, za_ref, zb_ref, da_ref, db_ref, h2_ref):
    deg = da_ref[...] + db_ref[...]
    inv = 1.0 / jnp.maximum(deg[:, 0:1], 1.0)
    h2_ref[...] = a2_ref[...] + (za_ref[...] + zb_ref[...]) * inv


def _tc3(a2, za, zb, da, db):
    row_spec = pl.BlockSpec((TC_R, D), lambda i: (i, 0))
    deg_spec = pl.BlockSpec((TC_R, DEGW), lambda i: (i, 0))
    return pl.pallas_call(
        _tc3_body,
        grid=(N // TC_R,),
        in_specs=[row_spec, row_spec, row_spec, deg_spec, deg_spec],
        out_specs=row_spec,
        out_shape=jax.ShapeDtypeStruct((N, D), jnp.float32),
    )(a2, za, zb, da, db)


def kernel(x, edge_index, pos_src_idx, pos_dst_idx, neg_src_idx, neg_dst_idx,
           W_self1, W_neigh1, b1, W_self2, W_neigh2, b2):
    src = edge_index[0]
    dst = edge_index[1]
    npad = EPAD - E
    srcp = jnp.concatenate(
        [src, N + (jnp.arange(npad, dtype=src.dtype) % 8)])
    dstp = jnp.concatenate(
        [dst, (jnp.arange(npad, dtype=dst.dtype) * 131) % N])
    zrows = jnp.zeros((8, D), jnp.float32)

    dega, degb = _sc_deg(dst)
    a1, y1 = _tc1(x, W_self1, W_neigh1, b1)
    z1a, z1b = _sc_segsum(jnp.concatenate([y1, zrows]), srcp, dstp)
    a2, y2 = _tc2(a1, z1a, z1b, dega, degb, W_self2, W_neigh2, b2)
    z2a, z2b = _sc_segsum(jnp.concatenate([y2, zrows]), srcp, dstp)
    h2 = _tc3(a2, z2a, z2b, dega, degb)

    cat_idx = jnp.concatenate(
        [pos_src_idx, pos_dst_idx, neg_src_idx, neg_dst_idx])
    out = _sc_gather(h2, cat_idx)
    return (out[0:B], out[B:2 * B], out[2 * B:3 * B], out[3 * B:4 * B])


# speedup vs baseline: 2.6114x; 1.1361x over previous
import jax
import jax.numpy as jnp
from jax import lax
from jax.experimental import pallas as pl
from jax.experimental.pallas import tpu as pltpu
import jax.experimental.pallas.tpu_sc as plsc

N = 10000
D = 128
E = 320000
B = 4096

NC = 2
NS = 16
NW = NC * NS
LANES = 16

K = 80
CPW = 125
NB = 2
NGRP = (CPW // NB // 2) * 2
TAIL = CPW - NGRP * NB
FR = 80
FCH = N // FR
FITERS = -(-FCH // NS)
DEGW = 128

TB = 4 * B
GPW = TB // NW
GK = 128
GCH = GPW // GK


def _mesh():
    return plsc.VectorSubcoreMesh(core_axis_name="c", subcore_axis_name="s",
                                  num_cores=NC, num_subcores=NS)


def _zero_2d(ref, rows, width):
    zero = jnp.zeros((LANES,), jnp.float32)

    def body(i, carry):
        for cb in range(width // LANES):
            ref[i, pl.ds(cb * LANES, LANES)] = zero
        return carry

    lax.fori_loop(0, rows, body, 0)


def _fill_ones(ref, rows, width):
    one = jnp.ones((LANES,), jnp.float32)

    def body(i, carry):
        for cb in range(width // LANES):
            ref[i, pl.ds(cb * LANES, LANES)] = one
        return carry

    lax.fori_loop(0, rows, body, 0)


def _flush_core_partial(c, s, src_s, a_out, b_out):
    for j in range(FITERS):
        fid = s + j * NS

        @pl.when(fid < FCH)
        def _():
            r0 = pl.multiple_of(fid * FR, 8)

            @pl.when(c == 0)
            def _():
                pltpu.sync_copy(src_s.at[pl.ds(r0, FR)],
                                a_out.at[pl.ds(r0, FR)])

            @pl.when(c == 1)
            def _():
                pltpu.sync_copy(src_s.at[pl.ds(r0, FR)],
                                b_out.at[pl.ds(r0, FR)])


def _sc_segsum_body(y_hbm, src_hbm, dst_hbm, za_out, zb_out,
                    srci_v, dsti_v, rows_v, acc_s,
                    sem_is, sem_id, sem_g, sem_s):
    c = lax.axis_index("c")
    s = lax.axis_index("s")
    wid = s * NC + c

    def issue_idx(j, iset, b):
        base = pl.multiple_of((wid * CPW + j) * K, 8)
        pltpu.async_copy(src_hbm.at[pl.ds(base, K)], srci_v.at[iset, b],
                         sem_is.at[iset, b])
        pltpu.async_copy(dst_hbm.at[pl.ds(base, K)], dsti_v.at[iset, b],
                         sem_id.at[iset, b])

    def wait_idx(j, iset, b):
        base = pl.multiple_of((wid * CPW + j) * K, 8)
        pltpu.make_async_copy(src_hbm.at[pl.ds(base, K)], srci_v.at[iset, b],
                              sem_is.at[iset, b]).wait()
        pltpu.make_async_copy(dst_hbm.at[pl.ds(base, K)], dsti_v.at[iset, b],
                              sem_id.at[iset, b]).wait()

    _zero_2d(rows_v.at[0], K, D)
    for j in range(FITERS):
        fid = s + j * NS

        @pl.when(fid < FCH)
        def _():
            r0 = pl.multiple_of(fid * FR, 8)
            pltpu.sync_copy(rows_v.at[0], acc_s.at[pl.ds(r0, FR)])
    plsc.subcore_barrier()

    for b in range(NB):
        issue_idx(b, 0, b)

    def dual(gg, carry):
        for half in range(2):
            g = 2 * gg + half

            @pl.when(g > 0)
            def _():
                for b in range(NB):
                    pltpu.make_async_copy(
                        rows_v.at[b], acc_s.at[dsti_v.at[1 - half, b]],
                        sem_s.at[b]).wait()

            gds = []
            for b in range(NB):
                j = g * NB + b
                wait_idx(j, half, b)
                gds.append(pltpu.async_copy(
                    y_hbm.at[srci_v.at[half, b]], rows_v.at[b], sem_g.at[b]))

            @pl.when(g + 1 < NGRP)
            def _():
                for b in range(NB):
                    issue_idx((g + 1) * NB + b, 1 - half, b)

            for b in range(NB):
                gds[b].wait()
                pltpu.async_copy(rows_v.at[b], acc_s.at[dsti_v.at[half, b]],
                                 sem_s.at[b], add=True)
        return carry

    lax.fori_loop(0, NGRP // 2, dual, 0)
    for b in range(NB):
        pltpu.make_async_copy(
            rows_v.at[b], acc_s.at[dsti_v.at[1, b]], sem_s.at[b]).wait()
    for t in range(TAIL):
        j = NGRP * NB + t
        base = pl.multiple_of((wid * CPW + j) * K, 8)
        pltpu.sync_copy(src_hbm.at[pl.ds(base, K)], srci_v.at[0, 0])
        pltpu.sync_copy(dst_hbm.at[pl.ds(base, K)], dsti_v.at[0, 0])
        pltpu.async_copy(y_hbm.at[srci_v.at[0, 0]], rows_v.at[0],
                         sem_g.at[0]).wait()
        pltpu.sync_copy(rows_v.at[0], acc_s.at[dsti_v.at[0, 0]], add=True)
    plsc.subcore_barrier()

    _flush_core_partial(c, s, acc_s, za_out, zb_out)


def _sc_segsum(y, src, dst):
    zshape = jax.ShapeDtypeStruct((N, D), jnp.float32)
    k = pl.kernel(
        _sc_segsum_body,
        out_type=[zshape, zshape],
        mesh=_mesh(),
        scratch_types=[
            pltpu.VMEM((2, NB, K), jnp.int32),
            pltpu.VMEM((2, NB, K), jnp.int32),
            pltpu.VMEM((NB, K, D), jnp.float32),
            pltpu.VMEM_SHARED((N, D), jnp.float32),
            pltpu.SemaphoreType.DMA((2, NB)),
            pltpu.SemaphoreType.DMA((2, NB)),
            pltpu.SemaphoreType.DMA((NB,)),
            pltpu.SemaphoreType.DMA((NB,)),
        ],
    )
    return k(y, src, dst)


def _sc_deg_body(dst_hbm, dega_out, degb_out,
                 dsti_v, ones_v, deg_s, sem_id, sem_d):
    c = lax.axis_index("c")
    s = lax.axis_index("s")
    wid = s * NC + c

    def issue_idx(j, iset, b):
        base = pl.multiple_of((wid * CPW + j) * K, 8)
        pltpu.async_copy(dst_hbm.at[pl.ds(base, K)], dsti_v.at[iset, b],
                         sem_id.at[iset, b])

    def wait_idx(j, iset, b):
        base = pl.multiple_of((wid * CPW + j) * K, 8)
        pltpu.make_async_copy(dst_hbm.at[pl.ds(base, K)], dsti_v.at[iset, b],
                              sem_id.at[iset, b]).wait()

    _zero_2d(ones_v, K, DEGW)
    for j in range(FITERS):
        fid = s + j * NS

        @pl.when(fid < FCH)
        def _():
            r0 = pl.multiple_of(fid * FR, 8)
            pltpu.sync_copy(ones_v, deg_s.at[pl.ds(r0, FR)])
    _fill_ones(ones_v, K, DEGW)
    plsc.subcore_barrier()

    for b in range(NB):
        issue_idx(b, 0, b)

    def dual(gg, carry):
        for half in range(2):
            g = 2 * gg + half

            @pl.when(g > 0)
            def _():
                for b in range(NB):
                    pltpu.make_async_copy(
                        ones_v, deg_s.at[dsti_v.at[1 - half, b]],
                        sem_d.at[b]).wait()

            for b in range(NB):
                j = g * NB + b
                wait_idx(j, half, b)
                pltpu.async_copy(ones_v, deg_s.at[dsti_v.at[half, b]],
                                 sem_d.at[b], add=True)

            @pl.when(g + 1 < NGRP)
            def _():
                for b in range(NB):
                    issue_idx((g + 1) * NB + b, 1 - half, b)
        return carry

    lax.fori_loop(0, NGRP // 2, dual, 0)
    for b in range(NB):
        pltpu.make_async_copy(
            ones_v, deg_s.at[dsti_v.at[1, b]], sem_d.at[b]).wait()
    for t in range(TAIL):
        j = NGRP * NB + t
        base = pl.multiple_of((wid * CPW + j) * K, 8)
        pltpu.sync_copy(dst_hbm.at[pl.ds(base, K)], dsti_v.at[0, 0])
        pltpu.sync_copy(ones_v, deg_s.at[dsti_v.at[0, 0]], add=True)
    plsc.subcore_barrier()

    _flush_core_partial(c, s, deg_s, dega_out, degb_out)


def _sc_deg(dst):
    dshape = jax.ShapeDtypeStruct((N, DEGW), jnp.float32)
    k = pl.kernel(
        _sc_deg_body,
        out_type=[dshape, dshape],
        mesh=_mesh(),
        scratch_types=[
            pltpu.VMEM((2, NB, K), jnp.int32),
            pltpu.VMEM((K, DEGW), jnp.float32),
            pltpu.VMEM_SHARED((N, DEGW), jnp.float32),
            pltpu.SemaphoreType.DMA((2, NB)),
            pltpu.SemaphoreType.DMA((NB,)),
        ],
    )
    return k(dst)


def _sc_gather_body(h_hbm, idx_hbm, out_hbm, idx_v, rows_v, sem):
    c = lax.axis_index("c")
    s = lax.axis_index("s")
    wid = s * NC + c
    for j in range(GCH):
        base = pl.multiple_of(wid * GPW + j * GK, 8)
        pltpu.sync_copy(idx_hbm.at[pl.ds(base, GK)], idx_v)
        pltpu.async_copy(h_hbm.at[idx_v], rows_v, sem).wait()
        pltpu.sync_copy(rows_v, out_hbm.at[pl.ds(base, GK)])


def _sc_gather(h, idx):
    k = pl.kernel(
        _sc_gather_body,
        out_type=jax.ShapeDtypeStruct((TB, D), jnp.float32),
        mesh=_mesh(),
        scratch_types=[
            pltpu.VMEM((GK,), jnp.int32),
            pltpu.VMEM((GK, D), jnp.float32),
            pltpu.SemaphoreType.DMA,
        ],
    )
    return k(h, idx)


TC_R = 1000


def _tc1_body(x_ref, ws_ref, wn_ref, b_ref, a1_ref, y1_ref):
    xv = x_ref[...]
    a1_ref[...] = (jnp.dot(xv, ws_ref[...], preferred_element_type=jnp.float32)
                   + b_ref[...])
    y1_ref[...] = jnp.dot(xv, wn_ref[...], preferred_element_type=jnp.float32)


def _tc1(x, ws, wn, b):
    row_spec = pl.BlockSpec((TC_R, D), lambda i: (i, 0))
    w_spec = pl.BlockSpec((D, D), lambda i: (0, 0))
    b_spec = pl.BlockSpec((1, D), lambda i: (0, 0))
    return pl.pallas_call(
        _tc1_body,
        grid=(N // TC_R,),
        in_specs=[row_spec, w_spec, w_spec, b_spec],
        out_specs=[row_spec, row_spec],
        out_shape=[jax.ShapeDtypeStruct((N, D), jnp.float32)] * 2,
    )(x, ws, wn, b.reshape(1, D))


def _tc2_body(a1_ref, za_ref, zb_ref, da_ref, db_ref, ws_ref, wn_ref, b_ref,
              a2_ref, y2_ref):
    deg = da_ref[...] + db_ref[...]
    inv = 1.0 / jnp.maximum(deg[:, 0:1], 1.0)
    h1 = jnp.maximum(a1_ref[...] + (za_ref[...] + zb_ref[...]) * inv, 0.0)
    a2_ref[...] = (jnp.dot(h1, ws_ref[...], preferred_element_type=jnp.float32)
                   + b_ref[...])
    y2_ref[...] = jnp.dot(h1, wn_ref[...], preferred_element_type=jnp.float32)


def _tc2(a1, za, zb, da, db, ws, wn, b):
    row_spec = pl.BlockSpec((TC_R, D), lambda i: (i, 0))
    deg_spec = pl.BlockSpec((TC_R, DEGW), lambda i: (i, 0))
    w_spec = pl.BlockSpec((D, D), lambda i: (0, 0))
    b_spec = pl.BlockSpec((1, D), lambda i: (0, 0))
    return pl.pallas_call(
        _tc2_body,
        grid=(N // TC_R,),
        in_specs=[row_spec, row_spec, row_spec, deg_spec, deg_spec,
                  w_spec, w_spec, b_spec],
        out_specs=[row_spec, row_spec],
        out_shape=[jax.ShapeDtypeStruct((N, D), jnp.float32)] * 2,
    )(a1, za, zb, da, db, ws, wn, b.reshape(1, D))


def _tc3_body(a2_ref, za_ref, zb_ref, da_ref, db_ref, h2_ref):
    deg = da_ref[...] + db_ref[...]
    inv = 1.0 / jnp.maximum(deg[:, 0:1], 1.0)
    h2_ref[...] = a2_ref[...] + (za_ref[...] + zb_ref[...]) * inv


def _tc3(a2, za, zb, da, db):
    row_spec = pl.BlockSpec((TC_R, D), lambda i: (i, 0))
    deg_spec = pl.BlockSpec((TC_R, DEGW), lambda i: (i, 0))
    return pl.pallas_call(
        _tc3_body,
        grid=(N // TC_R,),
        in_specs=[row_spec, row_spec, row_spec, deg_spec, deg_spec],
        out_specs=row_spec,
        out_shape=jax.ShapeDtypeStruct((N, D), jnp.float32),
    )(a2, za, zb, da, db)


def kernel(x, edge_index, pos_src_idx, pos_dst_idx, neg_src_idx, neg_dst_idx,
           W_self1, W_neigh1, b1, W_self2, W_neigh2, b2):
    src = edge_index[0]
    dst = edge_index[1]

    dega, degb = _sc_deg(dst)
    a1, y1 = _tc1(x, W_self1, W_neigh1, b1)
    z1a, z1b = _sc_segsum(y1, src, dst)
    a2, y2 = _tc2(a1, z1a, z1b, dega, degb, W_self2, W_neigh2, b2)
    z2a, z2b = _sc_segsum(y2, src, dst)
    h2 = _tc3(a2, z2a, z2b, dega, degb)

    cat_idx = jnp.concatenate(
        [pos_src_idx, pos_dst_idx, neg_src_idx, neg_dst_idx])
    out = _sc_gather(h2, cat_idx)
    return (out[0:B], out[B:2 * B], out[2 * B:3 * B], out[3 * B:4 * B])
